# Initial kernel scaffold; baseline (speedup 1.0000x reference)
import jax
import jax.numpy as jnp
from jax.experimental import pallas as pl


def kernel(action, neighbor_feature, W, b):
    raise NotImplementedError("write your pallas kernel here")


# speedup vs baseline: 20.5756x; 20.5756x over previous
import functools

import jax
import jax.numpy as jnp
from jax import lax
from jax.experimental import pallas as pl
from jax.experimental.pallas import tpu as pltpu
from jax.experimental.pallas import tpu_sc as plsc

_LANES = 16
_NC = 2
_NS = 16
_MAXK = 16
_GROUP = 16
_SPAN = _GROUP * _MAXK
_SPAN_BUF = _SPAN + 8


def _seg_ranges_body(action_ref, ends_ref, starts_ref):
    sn = action_ref[...] + 1
    bp = sn.shape[1]
    ends = sn
    k = 1
    while k < bp:
        shifted = jnp.concatenate(
            [jnp.zeros((1, k), jnp.int32), ends[:, :-k]], axis=1)
        ends = ends + shifted
        k *= 2
    ends_ref[...] = ends
    starts_ref[...] = ends - sn


def _make_sc_aggregate(n_rows, d_in, bp):
    segs = bp // _NS
    ngroups = segs // _GROUP
    dh = d_in // _NC

    def body(starts_hbm, ends_hbm, nf_hbm, out_hbm, starts_v, ends_v, rows_v, acc_v):
        c = lax.axis_index("c")
        s = lax.axis_index("s")
        seg0 = pl.multiple_of(s * segs, segs)
        d0 = pl.multiple_of(c * dh, dh)
        pltpu.sync_copy(starts_hbm.at[pl.ds(seg0, segs)], starts_v)
        pltpu.sync_copy(ends_hbm.at[pl.ds(seg0, segs)], ends_v)
        lane = lax.iota(jnp.int32, _LANES)

        def group_body(g, carry):
            gbase = g * _GROUP
            s16 = starts_v[pl.ds(gbase, _GROUP)]
            e16 = ends_v[pl.ds(gbase, _GROUP)]
            cnt = e16 - s16
            base = jnp.minimum((s16[0] // 8) * 8, n_rows - _SPAN_BUF)
            base = pl.multiple_of(base, 8)
            pltpu.sync_copy(
                nf_hbm.at[pl.ds(base, _SPAN_BUF), pl.ds(d0, dh)], rows_v)
            rel = s16 - base
            recip = 1.0 / jnp.maximum(cnt, 1).astype(jnp.float32)
            masks = [cnt > t for t in range(_MAXK)]
            rows_t = [jnp.where(masks[t], rel + t, 0) for t in range(_MAXK)]

            def d_body(d, carry2):
                col = jnp.zeros((_LANES,), jnp.int32) + d
                acc = jnp.zeros((_LANES,), jnp.float32)
                for t in range(_MAXK):
                    v = plsc.load_gather(
                        rows_v, [rows_t[t], col], mask=masks[t])
                    acc = acc + jnp.where(masks[t], v, 0.0)
                plsc.store_scatter(acc_v, [lane, col], acc * recip)
                return carry2

            lax.fori_loop(0, dh, d_body, 0)
            orow = pl.multiple_of(seg0 + gbase, _GROUP)
            pltpu.sync_copy(
                acc_v, out_hbm.at[pl.ds(orow, _GROUP), pl.ds(d0, dh)])
            return carry

        lax.fori_loop(0, ngroups, group_body, 0)

    mesh = plsc.VectorSubcoreMesh(
        core_axis_name="c", subcore_axis_name="s", num_cores=_NC,
        num_subcores=_NS)
    return functools.partial(
        pl.kernel,
        out_type=jax.ShapeDtypeStruct((bp, d_in), jnp.float32),
        mesh=mesh,
        compiler_params=pltpu.CompilerParams(needs_layout_passes=False),
        scratch_types=[
            pltpu.VMEM((segs,), jnp.int32),
            pltpu.VMEM((segs,), jnp.int32),
            pltpu.VMEM((_SPAN_BUF, dh), jnp.float32),
            pltpu.VMEM((_GROUP, dh), jnp.float32),
        ],
    )(body)


def _mm_body(x_ref, w_ref, b_ref, o_ref):
    o_ref[...] = (
        jnp.dot(x_ref[...], w_ref[...], preferred_element_type=jnp.float32)
        + b_ref[...])


def kernel(action, neighbor_feature, W, b):
    bsz = action.shape[0]
    n_rows, d_in = neighbor_feature.shape
    d_out = W.shape[1]

    block = _NS * _GROUP
    bp = ((bsz + block - 1) // block) * block

    ap = jnp.pad(action.astype(jnp.int32), (0, bp - bsz), constant_values=-1)
    ends, starts = pl.pallas_call(
        _seg_ranges_body,
        out_shape=(
            jax.ShapeDtypeStruct((1, bp), jnp.int32),
            jax.ShapeDtypeStruct((1, bp), jnp.int32),
        ),
    )(ap.reshape(1, bp))

    aggr = _make_sc_aggregate(n_rows, d_in, bp)(
        starts.reshape(bp), ends.reshape(bp), neighbor_feature)

    bm = 1024
    out = pl.pallas_call(
        _mm_body,
        grid=(bp // bm,),
        in_specs=[
            pl.BlockSpec((bm, d_in), lambda i: (i, 0)),
            pl.BlockSpec((d_in, d_out), lambda i: (0, 0)),
            pl.BlockSpec((1, d_out), lambda i: (0, 0)),
        ],
        out_specs=pl.BlockSpec((bm, d_out), lambda i: (i, 0)),
        out_shape=jax.ShapeDtypeStruct((bp, d_out), jnp.float32),
    )(aggr, W, b.reshape(1, d_out))

    return out[:bsz]
